# Initial kernel scaffold; baseline (speedup 1.0000x reference)
import jax
import jax.numpy as jnp
from jax.experimental import pallas as pl


def kernel(x, W, b):
    raise NotImplementedError("write your pallas kernel here")


# speedup vs baseline: 750.9209x; 750.9209x over previous
import functools

import jax
import jax.numpy as jnp
import numpy as np
from jax.experimental import pallas as pl
from jax.experimental.pallas import tpu as pltpu

_R = 2
_K = 3
_SAMPLES = 3


def _anchor_indices(H2: int, W2: int):
    xi = np.round(np.linspace(0.0, H2 - 1.0, _SAMPLES)).astype(np.int64)
    yi = np.round(np.linspace(0.0, W2 - 1.0, _SAMPLES)).astype(np.int64)
    xg, yg = np.meshgrid(xi, yi, indexing="ij")
    gather = (xg.flatten() * W2 + yg.flatten()).astype(np.int32)
    mask = (xg.flatten() * H2 + yg.flatten()).astype(np.int32)
    return gather, mask


def _body(x_ref, anch_ref, w_ref, b_ref, out_ref, y_s, *, flat, T, M, K):
    pid_n = pl.program_id(1)
    a = anch_ref[0]

    @pl.when(pid_n == 0)
    def _prep():
        for k in range(K):
            y_s[:, k * M:(k + 1) * M] = jnp.dot(
                w_ref[k], a, preferred_element_type=jnp.float32)

    xb = x_ref[0]
    nx = jnp.sum(xb * xb, axis=0)
    ny = jnp.sum(a * a, axis=0)
    inner = jax.lax.dot_general(
        a, xb, dimension_numbers=(((0,), (0,)), ((), ())),
        preferred_element_type=jnp.float32)
    d2 = jnp.maximum(nx[None, :] + ny[:, None] - 2.0 * inner, 0.0)

    tokr = pid_n * T + jax.lax.broadcasted_iota(jnp.int32, (1, T), 1)
    mask = jnp.concatenate(
        [(tokr == int(flat[m])).astype(jnp.float32) for m in range(M)], axis=0)
    d2 = jnp.where(mask > 0.0, -jnp.inf, d2)

    rows = [d2[m:m + 1] for m in range(M)]
    ranks = []
    for m in range(M):
        r = jnp.zeros_like(rows[m])
        for m2 in range(M):
            r += (rows[m2] < rows[m]).astype(jnp.float32)
            if m2 < m:
                r += (rows[m2] == rows[m]).astype(jnp.float32)
        ranks.append(r)
    rank = jnp.concatenate(ranks, axis=0)

    P = jnp.concatenate(
        [(rank == float(k)).astype(jnp.float32) for k in range(K)],
        axis=0)

    out_ref[0] = jnp.dot(y_s[...], P,
                         preferred_element_type=jnp.float32) + b_ref[...]


def kernel(x, W, b):
    B, C, H, Wd = x.shape
    H2, W2 = H // _R, Wd // _R
    C1 = C * _R * _R
    N = H2 * W2
    M = _SAMPLES * _SAMPLES
    gather_idx, flat = _anchor_indices(H2, W2)

    x2 = (x.reshape(B, C, H2, _R, W2, _R)
           .transpose(0, 1, 3, 5, 2, 4)
           .reshape(B, C1, N))
    anch = x2[:, :, gather_idx]
    Wr = jnp.transpose(W, (2, 0, 1))
    b2 = b.reshape(C1, 1)

    T = 512
    while N % T:
        T //= 2
    nb = N // T

    body = functools.partial(_body, flat=flat, T=T, M=M, K=_K)
    out = pl.pallas_call(
        body,
        grid=(B, nb),
        in_specs=[
            pl.BlockSpec((1, C1, T), lambda bi, ni: (bi, 0, ni)),
            pl.BlockSpec((1, C1, M), lambda bi, ni: (bi, 0, 0)),
            pl.BlockSpec((_K, C1, C1), lambda bi, ni: (0, 0, 0)),
            pl.BlockSpec((C1, 1), lambda bi, ni: (0, 0)),
        ],
        out_specs=pl.BlockSpec((1, C1, T), lambda bi, ni: (bi, 0, ni)),
        out_shape=jax.ShapeDtypeStruct((B, C1, N), jnp.float32),
        scratch_shapes=[pltpu.VMEM((C1, M * _K), jnp.float32)],
        compiler_params=pltpu.CompilerParams(
            dimension_semantics=("arbitrary", "arbitrary")),
    )(x2, anch, Wr, b2)

    return (out.reshape(B, C, _R, _R, H2, W2)
               .transpose(0, 1, 4, 2, 5, 3)
               .reshape(B, C, H, Wd))


# speedup vs baseline: 2458.2510x; 3.2736x over previous
import functools

import jax
import jax.numpy as jnp
import numpy as np
from jax.experimental import pallas as pl
from jax.experimental.pallas import tpu as pltpu

_R = 2
_K = 3
_SAMPLES = 3
_PAR = ((0, 0), (0, 1), (1, 0), (1, 1))


def _anchor_rc(H2: int, W2: int):
    xi = np.round(np.linspace(0.0, H2 - 1.0, _SAMPLES)).astype(np.int64)
    yi = np.round(np.linspace(0.0, W2 - 1.0, _SAMPLES)).astype(np.int64)
    xg, yg = np.meshgrid(xi, yi, indexing="ij")
    xf, yf = xg.flatten(), yg.flatten()
    mask = (xf * H2 + yf).astype(np.int32)
    return xf, yf, mask


def _body(x_ref, anch_ref, w_ref, b_ref, out_ref, y_s, *,
          flat, TI, W2, M, K, C):
    Wd = 2 * W2
    TT = TI * Wd
    pid_n = pl.program_id(1)
    a = anch_ref[0]

    @pl.when(pid_n == 0)
    def _prep():
        for k in range(K):
            y_s[:, k * M:(k + 1) * M] = jnp.dot(
                w_ref[k], a, preferred_element_type=jnp.float32)

    xv = x_ref[0]
    s0 = xv[:, :, 0:Wd].reshape(C, TT)
    s1 = xv[:, :, Wd:2 * Wd].reshape(C, TT)

    i_io = jax.lax.broadcasted_iota(jnp.int32, (1, TI, Wd), 1)
    w_io = jax.lax.broadcasted_iota(jnp.int32, (1, TI, Wd), 2)
    n_row = ((pid_n * TI + i_io) * W2 + (w_io >> 1)).reshape(1, TT)
    evenf = (((w_io & 1) == 0).astype(jnp.float32)).reshape(1, TT)
    oddf = 1.0 - evenf

    def dupsum(v):
        return v + jnp.where(evenf > 0.0,
                             jnp.roll(v, -1, axis=1), jnp.roll(v, 1, axis=1))

    e = jnp.sum(s0 * s0 + s1 * s1, axis=0)[None, :]
    nx = dupsum(e)

    def pinner(ag, s):
        return jax.lax.dot_general(
            ag, s, (((0,), (0,)), ((), ())),
            preferred_element_type=jnp.float32)

    i00 = pinner(a[0:C], s0)
    i01 = pinner(a[C:2 * C], s0)
    i10 = pinner(a[2 * C:3 * C], s1)
    i11 = pinner(a[3 * C:4 * C], s1)
    inner = dupsum((i00 + i10) * evenf + (i01 + i11) * oddf)

    ny = jnp.sum(a * a, axis=0)
    d2 = jnp.maximum(nx + ny[:, None] - 2.0 * inner, 0.0)

    mask = jnp.concatenate(
        [(n_row == int(flat[m])).astype(jnp.float32) for m in range(M)],
        axis=0)
    d2 = jnp.where(mask > 0.0, -jnp.inf, d2)

    rows = [d2[m:m + 1] for m in range(M)]
    ranks = []
    for m in range(M):
        r = jnp.zeros_like(rows[m])
        for m2 in range(M):
            r += (rows[m2] < rows[m]).astype(jnp.float32)
            if m2 < m:
                r += (rows[m2] == rows[m]).astype(jnp.float32)
        ranks.append(r)
    rank = jnp.concatenate(ranks, axis=0)

    P = jnp.concatenate(
        [(rank == float(k)).astype(jnp.float32) for k in range(K)],
        axis=0)
    PE = P * evenf
    PO = P * oddf

    bb = b_ref[...]
    for r1 in range(2):
        g0, g1 = 2 * r1, 2 * r1 + 1
        o = (jnp.dot(y_s[g0 * C:(g0 + 1) * C, :], PE,
                     preferred_element_type=jnp.float32)
             + jnp.dot(y_s[g1 * C:(g1 + 1) * C, :], PO,
                       preferred_element_type=jnp.float32))
        o = o + jnp.where(evenf > 0.0,
                          bb[g0 * C:(g0 + 1) * C], bb[g1 * C:(g1 + 1) * C])
        out_ref[0, :, :, r1 * Wd:(r1 + 1) * Wd] = o.reshape(C, TI, Wd)


def kernel(x, W, b):
    B, C, H, Wd = x.shape
    H2, W2 = H // _R, Wd // _R
    C1 = C * _R * _R
    M = _SAMPLES * _SAMPLES
    xf, yf, flat = _anchor_rc(H2, W2)

    perm = np.array([c * 4 + g for g in range(4) for c in range(C)])
    Wp = jnp.transpose(W[perm][:, perm, :], (2, 0, 1))
    bp = b[perm].reshape(C1, 1)
    anch = jnp.concatenate(
        [x[:, :, 2 * xf + r1, 2 * yf + r2] for (r1, r2) in _PAR],
        axis=1)

    x2r = x.reshape(B, C, H2, 2 * Wd)

    TI = 8
    while H2 % TI:
        TI //= 2
    nb = H2 // TI

    body = functools.partial(_body, flat=flat, TI=TI, W2=W2, M=M, K=_K, C=C)
    out = pl.pallas_call(
        body,
        grid=(B, nb),
        in_specs=[
            pl.BlockSpec((1, C, TI, 2 * Wd), lambda bi, ni: (bi, 0, ni, 0)),
            pl.BlockSpec((1, C1, M), lambda bi, ni: (bi, 0, 0)),
            pl.BlockSpec((_K, C1, C1), lambda bi, ni: (0, 0, 0)),
            pl.BlockSpec((C1, 1), lambda bi, ni: (0, 0)),
        ],
        out_specs=pl.BlockSpec((1, C, TI, 2 * Wd), lambda bi, ni: (bi, 0, ni, 0)),
        out_shape=jax.ShapeDtypeStruct((B, C, H2, 2 * Wd), jnp.float32),
        scratch_shapes=[pltpu.VMEM((C1, M * _K), jnp.float32)],
        compiler_params=pltpu.CompilerParams(
            dimension_semantics=("arbitrary", "arbitrary")),
    )(x2r, anch, Wp, bp)
    return out.reshape(B, C, H, Wd)


# speedup vs baseline: 2663.7920x; 1.0836x over previous
import functools

import jax
import jax.numpy as jnp
import numpy as np
from jax.experimental import pallas as pl
from jax.experimental.pallas import tpu as pltpu

_R = 2
_K = 3
_SAMPLES = 3
_PAR = ((0, 0), (0, 1), (1, 0), (1, 1))


def _anchor_rc(H2: int, W2: int):
    xi = np.round(np.linspace(0.0, H2 - 1.0, _SAMPLES)).astype(np.int64)
    yi = np.round(np.linspace(0.0, W2 - 1.0, _SAMPLES)).astype(np.int64)
    xg, yg = np.meshgrid(xi, yi, indexing="ij")
    xf, yf = xg.flatten(), yg.flatten()
    mask = (xf * H2 + yf).astype(np.int32)
    return xf, yf, mask


def _body(x_ref, anch_ref, w_ref, b_ref, out_ref, y_s, *,
          flat, TI, W2, M, K, C):
    Wd = 2 * W2
    TT = TI * Wd
    pid_n = pl.program_id(1)
    a = anch_ref[0]

    @pl.when(pid_n == 0)
    def _prep():
        for k in range(K):
            y_s[:, k * M:(k + 1) * M] = jnp.dot(
                w_ref[k], a, preferred_element_type=jnp.float32)

    xv = x_ref[0]
    s0 = xv[:, :, 0:Wd].reshape(C, TT)
    s1 = xv[:, :, Wd:2 * Wd].reshape(C, TT)

    i_io = jax.lax.broadcasted_iota(jnp.int32, (1, TI, Wd), 1)
    w_io = jax.lax.broadcasted_iota(jnp.int32, (1, TI, Wd), 2)
    n_row = ((pid_n * TI + i_io) * W2 + (w_io >> 1)).reshape(1, TT)
    evenf = (((w_io & 1) == 0).astype(jnp.float32)).reshape(1, TT)
    oddf = 1.0 - evenf

    def dupsum(v):
        return v + jnp.where(evenf > 0.0,
                             jnp.roll(v, -1, axis=1), jnp.roll(v, 1, axis=1))

    e = jnp.sum(s0 * s0 + s1 * s1, axis=0)[None, :]
    nx = dupsum(e)

    def pinner(ag, s):
        return jax.lax.dot_general(
            ag, s, (((0,), (0,)), ((), ())),
            preferred_element_type=jnp.float32)

    i00 = pinner(a[0:C], s0)
    i01 = pinner(a[C:2 * C], s0)
    i10 = pinner(a[2 * C:3 * C], s1)
    i11 = pinner(a[3 * C:4 * C], s1)
    inner = dupsum((i00 + i10) * evenf + (i01 + i11) * oddf)

    ny = jnp.sum(a * a, axis=0)
    d2 = jnp.maximum(nx + ny[:, None] - 2.0 * inner, 0.0)

    mask = jnp.concatenate(
        [(n_row == int(flat[m])).astype(jnp.float32) for m in range(M)],
        axis=0)
    d2 = jnp.where(mask > 0.0, -jnp.inf, d2)

    rows = [d2[m:m + 1] for m in range(M)]
    ranks = []
    for m in range(M):
        r = jnp.zeros_like(rows[m])
        for m2 in range(M):
            r += (rows[m2] < rows[m]).astype(jnp.float32)
            if m2 < m:
                r += (rows[m2] == rows[m]).astype(jnp.float32)
        ranks.append(r)
    rank = jnp.concatenate(ranks, axis=0)

    P = jnp.concatenate(
        [(rank == float(k)).astype(jnp.float32) for k in range(K)],
        axis=0)
    PE = P * evenf
    PO = P * oddf

    bb = b_ref[...]
    for r1 in range(2):
        g0, g1 = 2 * r1, 2 * r1 + 1
        o = (jnp.dot(y_s[g0 * C:(g0 + 1) * C, :], PE,
                     preferred_element_type=jnp.float32)
             + jnp.dot(y_s[g1 * C:(g1 + 1) * C, :], PO,
                       preferred_element_type=jnp.float32))
        o = o + jnp.where(evenf > 0.0,
                          bb[g0 * C:(g0 + 1) * C], bb[g1 * C:(g1 + 1) * C])
        out_ref[0, :, :, r1 * Wd:(r1 + 1) * Wd] = o.reshape(C, TI, Wd)


def kernel(x, W, b):
    B, C, H, Wd = x.shape
    H2, W2 = H // _R, Wd // _R
    C1 = C * _R * _R
    M = _SAMPLES * _SAMPLES
    xf, yf, flat = _anchor_rc(H2, W2)

    perm = np.array([c * 4 + g for g in range(4) for c in range(C)])
    Wp = jnp.transpose(W[perm][:, perm, :], (2, 0, 1))
    bp = b[perm].reshape(C1, 1)
    anch = jnp.concatenate(
        [jnp.concatenate(
            [jax.lax.slice(
                x,
                (0, 0, 2 * int(xf[m]) + r1, 2 * int(yf[m]) + r2),
                (B, C, 2 * int(xf[m]) + r1 + 1, 2 * int(yf[m]) + r2 + 1),
             ).reshape(B, C, 1) for m in range(M)],
            axis=2) for (r1, r2) in _PAR],
        axis=1)

    x2r = x.reshape(B, C, H2, 2 * Wd)

    TI = 8
    while H2 % TI:
        TI //= 2
    nb = H2 // TI

    body = functools.partial(_body, flat=flat, TI=TI, W2=W2, M=M, K=_K, C=C)
    out = pl.pallas_call(
        body,
        grid=(B, nb),
        in_specs=[
            pl.BlockSpec((1, C, TI, 2 * Wd), lambda bi, ni: (bi, 0, ni, 0)),
            pl.BlockSpec((1, C1, M), lambda bi, ni: (bi, 0, 0)),
            pl.BlockSpec((_K, C1, C1), lambda bi, ni: (0, 0, 0)),
            pl.BlockSpec((C1, 1), lambda bi, ni: (0, 0)),
        ],
        out_specs=pl.BlockSpec((1, C, TI, 2 * Wd), lambda bi, ni: (bi, 0, ni, 0)),
        out_shape=jax.ShapeDtypeStruct((B, C, H2, 2 * Wd), jnp.float32),
        scratch_shapes=[pltpu.VMEM((C1, M * _K), jnp.float32)],
        compiler_params=pltpu.CompilerParams(
            dimension_semantics=("arbitrary", "arbitrary")),
    )(x2r, anch, Wp, bp)
    return out.reshape(B, C, H, Wd)


# speedup vs baseline: 5176.3821x; 1.9432x over previous
import functools

import jax
import jax.numpy as jnp
import numpy as np
from jax.experimental import pallas as pl
from jax.experimental.pallas import tpu as pltpu

_R = 2
_K = 3
_SAMPLES = 3
_PAR = ((0, 0), (0, 1), (1, 0), (1, 1))


def _anchor_rc(H2: int, W2: int):
    xi = np.round(np.linspace(0.0, H2 - 1.0, _SAMPLES)).astype(np.int64)
    yi = np.round(np.linspace(0.0, W2 - 1.0, _SAMPLES)).astype(np.int64)
    xg, yg = np.meshgrid(xi, yi, indexing="ij")
    xf, yf = xg.flatten(), yg.flatten()
    mask = (xf * H2 + yf).astype(np.int32)
    return xf, yf, mask


def _body(x_ref, anch_ref, anch2_ref, w_ref, b_ref, out_ref, y_s, *,
          flat, HB, Wd, M, K, C):
    W2 = Wd // 2
    Q = HB * Wd
    pid_n = pl.program_id(1)
    a = anch_ref[0]

    @pl.when(pid_n == 0)
    def _prep():
        for k in range(K):
            full = jnp.dot(w_ref[k], a,
                           preferred_element_type=jnp.float32)
            for g in range(4):
                y_s[:, g * (K * M + 1) + k * M:
                       g * (K * M + 1) + (k + 1) * M] = \
                    full[g * C:(g + 1) * C, :]
        for g in range(4):
            y_s[:, g * (K * M + 1) + K * M:
                   g * (K * M + 1) + K * M + 1] = b_ref[g * C:(g + 1) * C, :]

    sflat = x_ref[0].reshape(C, Q)

    h_io = jax.lax.broadcasted_iota(jnp.int32, (1, HB, Wd), 1)
    w_io = jax.lax.broadcasted_iota(jnp.int32, (1, HB, Wd), 2)
    n_row = (((pid_n * HB + h_io) >> 1) * W2 + (w_io >> 1)).reshape(1, Q)
    ewf = ((w_io & 1) == 0).astype(jnp.float32).reshape(1, Q)
    ehf = ((h_io & 1) == 0).astype(jnp.float32).reshape(1, Q)
    ew = ewf > 0.0
    eh = ehf > 0.0
    mg = (ehf * ewf, ehf * (1.0 - ewf),
          (1.0 - ehf) * ewf, (1.0 - ehf) * (1.0 - ewf))

    def dupsum(v):
        v = v + jnp.where(ew, jnp.roll(v, -1, axis=1), jnp.roll(v, 1, axis=1))
        return v + jnp.where(eh, jnp.roll(v, -Wd, axis=1),
                             jnp.roll(v, Wd, axis=1))

    i_all = jax.lax.dot_general(
        anch2_ref[0], sflat, (((0,), (0,)), ((), ())),
        preferred_element_type=jnp.float32)
    pp = sum(i_all[g * M:(g + 1) * M] * mg[g] for g in range(4))
    inner = dupsum(pp)

    ny = jnp.sum(a * a, axis=0)
    d2 = ny[:, None] - 2.0 * inner

    maskc = jnp.concatenate(
        [(n_row == int(flat[m])).astype(jnp.float32) for m in range(M)],
        axis=0)
    d2 = jnp.where(maskc > 0.0, -jnp.inf, d2)

    sub_io = jax.lax.broadcasted_iota(jnp.int32, (M, Q), 0)
    rank = jnp.zeros((M, Q), jnp.float32)
    for m2 in range(M):
        row = d2[m2:m2 + 1]
        rank = rank + (row < d2).astype(jnp.float32)
        rank = rank + ((row == d2).astype(jnp.float32)
                       * (sub_io > m2).astype(jnp.float32))

    pk = [(rank == float(k)).astype(jnp.float32) for k in range(K)]
    pieces = []
    for g in range(4):
        for k in range(K):
            pieces.append(pk[k] * mg[g])
        pieces.append(mg[g])
    pstack = jnp.concatenate(pieces, axis=0)

    out = jnp.dot(y_s[...], pstack,
                  preferred_element_type=jnp.float32)
    out_ref[0] = out.reshape(C, HB, Wd)


def kernel(x, W, b):
    B, C, H, Wd = x.shape
    H2, W2 = H // _R, Wd // _R
    C1 = C * _R * _R
    M = _SAMPLES * _SAMPLES
    xf, yf, flat = _anchor_rc(H2, W2)

    perm = np.array([c * 4 + g for g in range(4) for c in range(C)])
    Wp = jnp.transpose(W[perm][:, perm, :], (2, 0, 1))
    bp = b[perm].reshape(C1, 1)

    anch = jnp.concatenate(
        [jnp.concatenate(
            [jax.lax.slice(
                x,
                (0, 0, 2 * int(xf[m]) + r1, 2 * int(yf[m]) + r2),
                (B, C, 2 * int(xf[m]) + r1 + 1, 2 * int(yf[m]) + r2 + 1),
             ).reshape(B, C, 1) for m in range(M)],
            axis=2) for (r1, r2) in _PAR],
        axis=1)
    anch2 = jnp.transpose(anch.reshape(B, 4, C, M), (0, 2, 1, 3)) \
               .reshape(B, C, 4 * M)

    HB = 16
    while H % HB:
        HB //= 2
    nb = H // HB

    body = functools.partial(_body, flat=flat, HB=HB, Wd=Wd, M=M, K=_K, C=C)
    return pl.pallas_call(
        body,
        grid=(B, nb),
        in_specs=[
            pl.BlockSpec((1, C, HB, Wd), lambda bi, ni: (bi, 0, ni, 0)),
            pl.BlockSpec((1, C1, M), lambda bi, ni: (bi, 0, 0)),
            pl.BlockSpec((1, C, 4 * M), lambda bi, ni: (bi, 0, 0)),
            pl.BlockSpec((_K, C1, C1), lambda bi, ni: (0, 0, 0)),
            pl.BlockSpec((C1, 1), lambda bi, ni: (0, 0)),
        ],
        out_specs=pl.BlockSpec((1, C, HB, Wd), lambda bi, ni: (bi, 0, ni, 0)),
        out_shape=jax.ShapeDtypeStruct((B, C, H, Wd), jnp.float32),
        scratch_shapes=[pltpu.VMEM((C, 4 * (_K * M + 1)), jnp.float32)],
        compiler_params=pltpu.CompilerParams(
            dimension_semantics=("arbitrary", "arbitrary")),
    )(x, anch, anch2, Wp, bp)
